# Initial kernel scaffold; baseline (speedup 1.0000x reference)
import jax
import jax.numpy as jnp
from jax.experimental import pallas as pl


def kernel(node_features, edge_features, context_features, edge_index, params):
    raise NotImplementedError("write your pallas kernel here")


# speedup vs baseline: 4.2867x; 4.2867x over previous
import jax
import jax.numpy as jnp
from jax import lax
from jax.experimental import pallas as pl
from jax.experimental.pallas import tpu as pltpu
from jax.experimental.pallas import tpu_sc as plsc

F32 = jnp.float32
N_NODES = 10000
N_EDGES = 160000
D = 128
E_BLK = 4000
GW = 128
N_SUBCORES = 16
NPAD = 10240
ROWS_PER_SUB = NPAD // N_SUBCORES

_LN_EPS = 1e-6


def _ln(x, g, b):
    mu = jnp.mean(x, axis=-1, keepdims=True)
    xc = x - mu
    var = jnp.mean(xc * xc, axis=-1, keepdims=True)
    return xc * lax.rsqrt(var + _LN_EPS) * g + b


def _mlp_body(has_ln, *refs):
    if has_ln:
        x_ref, w1, b1, w2, b2, w3, b3, g, be, o_ref = refs
    else:
        x_ref, w1, b1, w2, b2, w3, b3, o_ref = refs
    h = jnp.maximum(x_ref[...] @ w1[...] + b1[...], 0.0)
    h = jnp.maximum(h @ w2[...] + b2[...], 0.0)
    h = h @ w3[...] + b3[...]
    if has_ln:
        h = _ln(h, g[...], be[...])
    o_ref[...] = h


def _mlp_rows(x, mp, blk=None):
    (w1, b1), (w2, b2), (w3, b3) = mp["layers"]
    ln = mp["ln"]
    n, din = x.shape
    dout = w3.shape[1]
    if blk is None:
        blk = n
    grid = n // blk
    full = lambda a: pl.BlockSpec(a.shape, lambda i: (0,) * a.ndim)
    args = [x, w1, b1.reshape(1, -1), w2, b2.reshape(1, -1),
            w3, b3.reshape(1, -1)]
    if ln is not None:
        args += [ln[0].reshape(1, -1), ln[1].reshape(1, -1)]
    in_specs = [pl.BlockSpec((blk, din), lambda i: (i, 0))]
    in_specs += [full(a) for a in args[1:]]
    body = lambda *refs: _mlp_body(ln is not None, *refs)
    return pl.pallas_call(
        body,
        grid=(grid,),
        in_specs=in_specs,
        out_specs=pl.BlockSpec((blk, dout), lambda i: (i, 0)),
        out_shape=jax.ShapeDtypeStruct((n, dout), F32),
    )(*args)


def _pre_body(nl_ref, ctx_ref, wsrc, wdst, wec, be1, wnc, bn1,
              gsrc_ref, gdst_ref, bee_ref, bne_ref):
    nl = nl_ref[...]
    gsrc_ref[...] = nl @ wsrc[...]
    gdst_ref[...] = nl @ wdst[...]
    c = ctx_ref[...]
    bee_ref[...] = c @ wec[...] + be1[...]
    bne_ref[...] = c @ wnc[...] + bn1[...]


def _precompute(node_lat, ctx_lat, w1e, b1e, w1n, b1n):
    args = [node_lat, ctx_lat, w1e[128:256], w1e[256:384], w1e[384:512],
            b1e.reshape(1, -1), w1n[256:384], b1n.reshape(1, -1)]
    return pl.pallas_call(
        _pre_body,
        out_shape=(jax.ShapeDtypeStruct((N_NODES, D), F32),
                   jax.ShapeDtypeStruct((N_NODES, D), F32),
                   jax.ShapeDtypeStruct((1, D), F32),
                   jax.ShapeDtypeStruct((1, D), F32)),
    )(*args)


def _edge_step_body(el_ref, gs_ref, gd_ref, w1, beff, w2, b2, w3, b3, g, be,
                    ne_ref, elo_ref):
    el = el_ref[...]
    h = el @ w1[...] + gs_ref[...] + gd_ref[...] + beff[...]
    h = jnp.maximum(h, 0.0)
    h = jnp.maximum(h @ w2[...] + b2[...], 0.0)
    h = h @ w3[...] + b3[...]
    ne = _ln(h, g[...], be[...])
    ne_ref[...] = ne
    elo_ref[...] = el + ne


def _edge_step(edge_lat, gs, gd, w1_edge, bee, mp):
    (_, _), (w2, b2), (w3, b3) = mp["layers"]
    g, be = mp["ln"]
    args = [edge_lat, gs, gd, w1_edge, bee, w2, b2.reshape(1, -1),
            w3, b3.reshape(1, -1), g.reshape(1, -1), be.reshape(1, -1)]
    blk = pl.BlockSpec((E_BLK, D), lambda i: (i, 0))
    full = lambda a: pl.BlockSpec(a.shape, lambda i: (0,) * a.ndim)
    return pl.pallas_call(
        _edge_step_body,
        grid=(N_EDGES // E_BLK,),
        in_specs=[blk, blk, blk] + [full(a) for a in args[3:]],
        out_specs=[blk, blk],
        out_shape=(jax.ShapeDtypeStruct((N_EDGES, D), F32),
                   jax.ShapeDtypeStruct((N_EDGES, D), F32)),
    )(*args)


def _node_ctx_body(nl_ref, p2_ref, ctx_ref, wnl, wnp, bneff, wn2, bn2,
                   wn3, bn3, gn, ben, wcc, wcn, wce, bc1, wc2, bc2,
                   wc3, bc3, gc, bec, nlo_ref, ctxo_ref):
    nl = nl_ref[...]
    pooled = (p2_ref[0] + p2_ref[1])[:N_NODES]
    h = jnp.maximum(nl @ wnl[...] + pooled @ wnp[...] + bneff[...], 0.0)
    h = jnp.maximum(h @ wn2[...] + bn2[...], 0.0)
    h = h @ wn3[...] + bn3[...]
    nn = _ln(h, gn[...], ben[...])
    nlo_ref[...] = nl + nn
    snn = jnp.sum(nn, axis=0, keepdims=True)
    sne = jnp.sum(pooled, axis=0, keepdims=True)
    c = ctx_ref[...]
    hc = jnp.maximum(c @ wcc[...] + snn @ wcn[...] + sne @ wce[...]
                     + bc1[...], 0.0)
    hc = jnp.maximum(hc @ wc2[...] + bc2[...], 0.0)
    hc = hc @ wc3[...] + bc3[...]
    ctxo_ref[...] = c + _ln(hc, gc[...], bec[...])


def _node_ctx_step(node_lat, pooled2, ctx_lat, w1n, bne, node_mp, ctx_mp):
    (_, _), (wn2, bn2), (wn3, bn3) = node_mp["layers"]
    gn, ben = node_mp["ln"]
    (wc1, bc1), (wc2, bc2), (wc3, bc3) = ctx_mp["layers"]
    gc, bec = ctx_mp["ln"]
    args = [node_lat, pooled2, ctx_lat,
            w1n[0:128], w1n[128:256], bne,
            wn2, bn2.reshape(1, -1), wn3, bn3.reshape(1, -1),
            gn.reshape(1, -1), ben.reshape(1, -1),
            wc1[0:128], wc1[128:256], wc1[256:384], bc1.reshape(1, -1),
            wc2, bc2.reshape(1, -1), wc3, bc3.reshape(1, -1),
            gc.reshape(1, -1), bec.reshape(1, -1)]
    return pl.pallas_call(
        _node_ctx_body,
        out_shape=(jax.ShapeDtypeStruct((N_NODES, D), F32),
                   jax.ShapeDtypeStruct((1, D), F32)),
    )(*args)


_SC_MESH = plsc.VectorSubcoreMesh(core_axis_name="core",
                                  subcore_axis_name="subcore")


def _sc_gather(gsrc, gdst, src_idx, dst_idx):
    @pl.kernel(out_type=(jax.ShapeDtypeStruct((N_EDGES, D), F32),
                         jax.ShapeDtypeStruct((N_EDGES, D), F32)),
               mesh=_SC_MESH)
    def k(gsrc_hbm, gdst_hbm, si_hbm, di_hbm, os_hbm, od_hbm):
        def body(si_vmem, di_vmem, os_vmem, od_vmem):
            pltpu.sync_copy(gsrc_hbm.at[si_vmem.at[0]], os_vmem)
            pltpu.sync_copy(gdst_hbm.at[di_vmem.at[0]], od_vmem)

        pltpu.emit_pipeline(
            body,
            grid=(N_EDGES // GW,),
            in_specs=[pl.BlockSpec((1, GW), lambda i: (0, i)),
                      pl.BlockSpec((1, GW), lambda i: (0, i))],
            out_specs=[pl.BlockSpec((GW, D), lambda i: (i, 0)),
                       pl.BlockSpec((GW, D), lambda i: (i, 0))],
            core_axis_name=("core", "subcore"),
            dimension_semantics=(pltpu.PARALLEL,),
        )(si_hbm, di_hbm, os_hbm, od_hbm)

    return k(gsrc, gdst, src_idx, dst_idx)


def _sc_segment_sum(new_edge, dst_idx, zeros):
    @pl.kernel(out_type=jax.ShapeDtypeStruct((2, NPAD, D), F32),
               mesh=_SC_MESH,
               scratch_types=[pltpu.VMEM_SHARED((NPAD, D), F32)])
    def k(ne_hbm, di_hbm, z_hbm, o_hbm, acc):
        cid = lax.axis_index("core")
        sid = lax.axis_index("subcore")
        rows = pl.ds(sid * ROWS_PER_SUB, ROWS_PER_SUB)
        pltpu.sync_copy(z_hbm.at[rows], acc.at[rows])
        plsc.subcore_barrier()

        def body(di_vmem, ne_vmem):
            pltpu.sync_copy(ne_vmem, acc.at[di_vmem.at[0]], add=True)

        pltpu.emit_pipeline(
            body,
            grid=(N_EDGES // GW,),
            in_specs=[pl.BlockSpec((1, GW), lambda i: (0, i)),
                      pl.BlockSpec((GW, D), lambda i: (i, 0))],
            out_specs=[],
            core_axis_name=("core", "subcore"),
            dimension_semantics=(pltpu.PARALLEL,),
        )(di_hbm, ne_hbm)
        plsc.subcore_barrier()
        pltpu.sync_copy(acc.at[rows], o_hbm.at[cid, rows])

    return k(new_edge, dst_idx, zeros)


def kernel(node_features, edge_features, context_features, edge_index,
           params):
    p = params
    src = edge_index[0].reshape(1, N_EDGES)
    dst = edge_index[1].reshape(1, N_EDGES)
    zeros = jnp.zeros((NPAD, D), F32)

    node_lat = _mlp_rows(node_features, p["enc_node"])
    edge_lat = _mlp_rows(edge_features, p["enc_edge"], blk=E_BLK)
    ctx_lat = _mlp_rows(context_features, p["enc_ctx"])

    for s in range(2):
        sp = p["proc"][s]
        w1e, b1e = sp["edge"]["layers"][0]
        w1n, b1n = sp["node"]["layers"][0]
        gsrc, gdst, bee, bne = _precompute(node_lat, ctx_lat,
                                           w1e, b1e, w1n, b1n)
        gs, gd = _sc_gather(gsrc, gdst, src, dst)
        new_edge, edge_lat = _edge_step(edge_lat, gs, gd, w1e[0:128], bee,
                                        sp["edge"])
        pooled2 = _sc_segment_sum(new_edge, dst, zeros)
        node_lat, ctx_lat = _node_ctx_step(node_lat, pooled2, ctx_lat,
                                           w1n, bne, sp["node"], sp["ctx"])

    node_out = _mlp_rows(node_lat, p["dec_node"])
    edge_out = _mlp_rows(edge_lat, p["dec_edge"], blk=E_BLK)
    ctx_out = _mlp_rows(ctx_lat, p["dec_ctx"])
    return (node_out, edge_out, ctx_out)


# speedup vs baseline: 4.8400x; 1.1291x over previous
import jax
import jax.numpy as jnp
from jax import lax
from jax.experimental import pallas as pl
from jax.experimental.pallas import tpu as pltpu
from jax.experimental.pallas import tpu_sc as plsc

F32 = jnp.float32
N_NODES = 10000
N_EDGES = 160000
D = 128
E_BLK = 4000
GW = 128
N_SUBCORES = 16
NPAD = 10240
ROWS_PER_SUB = NPAD // N_SUBCORES

_LN_EPS = 1e-6


def _ln(x, g, b):
    mu = jnp.mean(x, axis=-1, keepdims=True)
    xc = x - mu
    var = jnp.mean(xc * xc, axis=-1, keepdims=True)
    return xc * lax.rsqrt(var + _LN_EPS) * g + b


def _mlp3(x, w1, b1, w2, b2, w3, b3):
    h = jnp.maximum(x @ w1[...] + b1[...], 0.0)
    h = jnp.maximum(h @ w2[...] + b2[...], 0.0)
    return h @ w3[...] + b3[...]


def _flat(mp):
    (w1, b1), (w2, b2), (w3, b3) = mp["layers"]
    out = [w1, b1.reshape(1, -1), w2, b2.reshape(1, -1), w3, b3.reshape(1, -1)]
    if mp["ln"] is not None:
        out += [mp["ln"][0].reshape(1, -1), mp["ln"][1].reshape(1, -1)]
    return out


def _mlp_body(has_ln, *refs):
    if has_ln:
        x_ref, w1, b1, w2, b2, w3, b3, g, be, o_ref = refs
    else:
        x_ref, w1, b1, w2, b2, w3, b3, o_ref = refs
    h = _mlp3(x_ref[...], w1, b1, w2, b2, w3, b3)
    if has_ln:
        h = _ln(h, g[...], be[...])
    o_ref[...] = h


def _mlp_rows(x, mp):
    n = x.shape[0]
    dout = mp["layers"][2][0].shape[1]
    ln = mp["ln"]
    body = lambda *refs: _mlp_body(ln is not None, *refs)
    return pl.pallas_call(
        body, out_shape=jax.ShapeDtypeStruct((n, dout), F32),
    )(x, *_flat(mp))


def _enc_pre_body(nf, ctx, w1, b1, w2, b2, w3, b3, g, be,
                  wsrc, wdst, wec, be1, wnc, bn1,
                  nl_o, gsrc_o, gdst_o, bee_o, bne_o):
    nl = _ln(_mlp3(nf[...], w1, b1, w2, b2, w3, b3), g[...], be[...])
    nl_o[...] = nl
    gsrc_o[...] = nl @ wsrc[...]
    gdst_o[...] = nl @ wdst[...]
    c = ctx[...]
    bee_o[...] = c @ wec[...] + be1[...]
    bne_o[...] = c @ wnc[...] + bn1[...]


def _enc_node_pre(node_features, ctx_lat, enc_mp, w1e, b1e, w1n, b1n):
    args = [node_features, ctx_lat] + _flat(enc_mp) + [
        w1e[128:256], w1e[256:384], w1e[384:512], b1e.reshape(1, -1),
        w1n[256:384], b1n.reshape(1, -1)]
    return pl.pallas_call(
        _enc_pre_body,
        out_shape=(jax.ShapeDtypeStruct((N_NODES, D), F32),
                   jax.ShapeDtypeStruct((N_NODES, D), F32),
                   jax.ShapeDtypeStruct((N_NODES, D), F32),
                   jax.ShapeDtypeStruct((1, D), F32),
                   jax.ShapeDtypeStruct((1, D), F32)),
    )(*args)


def _edge0_body(ef, gs, gd, ew1, eb1, ew2, eb2, ew3, eb3, eg, ebe,
                w1, bee, w2, b2, w3, b3, g, be, ne_o, el_o):
    el = _ln(_mlp3(ef[...], ew1, eb1, ew2, eb2, ew3, eb3), eg[...], ebe[...])
    h = jnp.maximum(el @ w1[...] + gs[...] + gd[...] + bee[...], 0.0)
    h = jnp.maximum(h @ w2[...] + b2[...], 0.0)
    h = h @ w3[...] + b3[...]
    ne = _ln(h, g[...], be[...])
    ne_o[...] = ne
    el_o[...] = el + ne


def _edge_step0(edge_features, gs, gd, enc_mp, w1e, bee, mp):
    args = ([edge_features, gs, gd] + _flat(enc_mp)
            + [w1e[0:128], bee] + _flat(mp)[2:])
    blk = pl.BlockSpec((E_BLK, D), lambda i: (i, 0))
    blk16 = pl.BlockSpec((E_BLK, 16), lambda i: (i, 0))
    full = lambda a: pl.BlockSpec(a.shape, lambda i: (0,) * a.ndim)
    return pl.pallas_call(
        _edge0_body,
        grid=(N_EDGES // E_BLK,),
        in_specs=[blk16, blk, blk] + [full(a) for a in args[3:]],
        out_specs=[blk, blk],
        out_shape=(jax.ShapeDtypeStruct((N_EDGES, D), F32),
                   jax.ShapeDtypeStruct((N_EDGES, D), F32)),
    )(*args)


def _edge1_body(el_ref, gs, gd, w1, bee, w2, b2, w3, b3, g, be,
                dw1, db1, dw2, db2, dw3, db3, ne_o, eo_o):
    el = el_ref[...]
    h = jnp.maximum(el @ w1[...] + gs[...] + gd[...] + bee[...], 0.0)
    h = jnp.maximum(h @ w2[...] + b2[...], 0.0)
    h = h @ w3[...] + b3[...]
    ne = _ln(h, g[...], be[...])
    ne_o[...] = ne
    eo_o[...] = _mlp3(el + ne, dw1, db1, dw2, db2, dw3, db3)


def _edge_step1(edge_lat, gs, gd, w1e, bee, mp, dec_mp):
    args = ([edge_lat, gs, gd, w1e[0:128], bee]
            + _flat(mp)[2:] + _flat(dec_mp))
    blk = pl.BlockSpec((E_BLK, D), lambda i: (i, 0))
    full = lambda a: pl.BlockSpec(a.shape, lambda i: (0,) * a.ndim)
    return pl.pallas_call(
        _edge1_body,
        grid=(N_EDGES // E_BLK,),
        in_specs=[blk, blk, blk] + [full(a) for a in args[3:]],
        out_specs=[blk, blk],
        out_shape=(jax.ShapeDtypeStruct((N_EDGES, D), F32),
                   jax.ShapeDtypeStruct((N_EDGES, D), F32)),
    )(*args)


def _node_core(nl_ref, p2_ref, ctx_ref, wnl, wnp, bneff, wn2, bn2, wn3, bn3,
               gn, ben, wcc, wcn, wce, bc1, wc2, bc2, wc3, bc3, gc, bec):
    nl = nl_ref[...]
    pooled = (p2_ref[0] + p2_ref[1])[:N_NODES]
    h = jnp.maximum(nl @ wnl[...] + pooled @ wnp[...] + bneff[...], 0.0)
    h = jnp.maximum(h @ wn2[...] + bn2[...], 0.0)
    h = h @ wn3[...] + bn3[...]
    nn = _ln(h, gn[...], ben[...])
    snn = jnp.sum(nn, axis=0, keepdims=True)
    sne = jnp.sum(pooled, axis=0, keepdims=True)
    c = ctx_ref[...]
    hc = jnp.maximum(c @ wcc[...] + snn @ wcn[...] + sne @ wce[...]
                     + bc1[...], 0.0)
    hc = jnp.maximum(hc @ wc2[...] + bc2[...], 0.0)
    hc = hc @ wc3[...] + bc3[...]
    return nl + nn, c + _ln(hc, gc[...], bec[...])


def _node_pre_body(*refs):
    (core, (wsrc, wdst, wec, be1, wnc, bn1),
     (nlo, ctxo, gsrc_o, gdst_o, bee_o, bne_o)) = refs[:22], refs[22:28], refs[28:]
    nl_new, ctx_new = _node_core(*core)
    nlo[...] = nl_new
    ctxo[...] = ctx_new
    gsrc_o[...] = nl_new @ wsrc[...]
    gdst_o[...] = nl_new @ wdst[...]
    bee_o[...] = ctx_new @ wec[...] + be1[...]
    bne_o[...] = ctx_new @ wnc[...] + bn1[...]


def _node_dec_body(*refs):
    (core, (ndw1, ndb1, ndw2, ndb2, ndw3, ndb3,
            cdw1, cdb1, cdw2, cdb2, cdw3, cdb3),
     (no_o, co_o)) = refs[:22], refs[22:34], refs[34:]
    nl_new, ctx_new = _node_core(*core)
    no_o[...] = _mlp3(nl_new, ndw1, ndb1, ndw2, ndb2, ndw3, ndb3)
    co_o[...] = _mlp3(ctx_new, cdw1, cdb1, cdw2, cdb2, cdw3, cdb3)


def _node_core_args(node_lat, pooled2, ctx_lat, w1n, bne, node_mp, ctx_mp):
    wc1 = ctx_mp["layers"][0][0]
    bc1 = ctx_mp["layers"][0][1]
    return ([node_lat, pooled2, ctx_lat, w1n[0:128], w1n[128:256], bne]
            + _flat(node_mp)[2:]
            + [wc1[0:128], wc1[128:256], wc1[256:384], bc1.reshape(1, -1)]
            + _flat(ctx_mp)[2:])


def _node_step_pre(node_lat, pooled2, ctx_lat, w1n, bne, node_mp, ctx_mp,
                   w1e_n, b1e_n, w1n_n, b1n_n):
    args = _node_core_args(node_lat, pooled2, ctx_lat, w1n, bne,
                           node_mp, ctx_mp) + [
        w1e_n[128:256], w1e_n[256:384], w1e_n[384:512],
        b1e_n.reshape(1, -1), w1n_n[256:384], b1n_n.reshape(1, -1)]
    return pl.pallas_call(
        _node_pre_body,
        out_shape=(jax.ShapeDtypeStruct((N_NODES, D), F32),
                   jax.ShapeDtypeStruct((1, D), F32),
                   jax.ShapeDtypeStruct((N_NODES, D), F32),
                   jax.ShapeDtypeStruct((N_NODES, D), F32),
                   jax.ShapeDtypeStruct((1, D), F32),
                   jax.ShapeDtypeStruct((1, D), F32)),
    )(*args)


def _node_step_dec(node_lat, pooled2, ctx_lat, w1n, bne, node_mp, ctx_mp,
                   dec_node_mp, dec_ctx_mp):
    args = (_node_core_args(node_lat, pooled2, ctx_lat, w1n, bne,
                            node_mp, ctx_mp)
            + _flat(dec_node_mp) + _flat(dec_ctx_mp))
    return pl.pallas_call(
        _node_dec_body,
        out_shape=(jax.ShapeDtypeStruct((N_NODES, D), F32),
                   jax.ShapeDtypeStruct((1, D), F32)),
    )(*args)


_SC_MESH = plsc.VectorSubcoreMesh(core_axis_name="core",
                                  subcore_axis_name="subcore")


def _sc_gather(gsrc, gdst, src_idx, dst_idx):
    @pl.kernel(out_type=(jax.ShapeDtypeStruct((N_EDGES, D), F32),
                         jax.ShapeDtypeStruct((N_EDGES, D), F32)),
               mesh=_SC_MESH,
               scratch_types=[pltpu.SemaphoreType.DMA,
                              pltpu.SemaphoreType.DMA])
    def k(gsrc_hbm, gdst_hbm, si_hbm, di_hbm, os_hbm, od_hbm, sem1, sem2):
        def body(si_vmem, di_vmem, os_vmem, od_vmem):
            c1 = pltpu.async_copy(gsrc_hbm.at[si_vmem.at[0]], os_vmem, sem1)
            c2 = pltpu.async_copy(gdst_hbm.at[di_vmem.at[0]], od_vmem, sem2)
            c1.wait()
            c2.wait()

        pltpu.emit_pipeline(
            body,
            grid=(N_EDGES // GW,),
            in_specs=[pl.BlockSpec((1, GW), lambda i: (0, i)),
                      pl.BlockSpec((1, GW), lambda i: (0, i))],
            out_specs=[pl.BlockSpec((GW, D), lambda i: (i, 0)),
                       pl.BlockSpec((GW, D), lambda i: (i, 0))],
            core_axis_name=("core", "subcore"),
            dimension_semantics=(pltpu.PARALLEL,),
        )(si_hbm, di_hbm, os_hbm, od_hbm)

    return k(gsrc, gdst, src_idx, dst_idx)


def _sc_segment_sum(new_edge, dst_idx, zeros):
    @pl.kernel(out_type=jax.ShapeDtypeStruct((2, NPAD, D), F32),
               mesh=_SC_MESH,
               scratch_types=[pltpu.VMEM_SHARED((NPAD, D), F32)])
    def k(ne_hbm, di_hbm, z_hbm, o_hbm, acc):
        cid = lax.axis_index("core")
        sid = lax.axis_index("subcore")
        rows = pl.ds(sid * ROWS_PER_SUB, ROWS_PER_SUB)
        pltpu.sync_copy(z_hbm.at[rows], acc.at[rows])
        plsc.subcore_barrier()

        def body(di_vmem, ne_vmem):
            pltpu.sync_copy(ne_vmem, acc.at[di_vmem.at[0]], add=True)

        pltpu.emit_pipeline(
            body,
            grid=(N_EDGES // GW,),
            in_specs=[pl.BlockSpec((1, GW), lambda i: (0, i)),
                      pl.BlockSpec((GW, D), lambda i: (i, 0))],
            out_specs=[],
            core_axis_name=("core", "subcore"),
            dimension_semantics=(pltpu.PARALLEL,),
        )(di_hbm, ne_hbm)
        plsc.subcore_barrier()
        pltpu.sync_copy(acc.at[rows], o_hbm.at[cid, rows])

    return k(new_edge, dst_idx, zeros)


def kernel(node_features, edge_features, context_features, edge_index,
           params):
    p = params
    src = edge_index[0].reshape(1, N_EDGES)
    dst = edge_index[1].reshape(1, N_EDGES)
    zeros = jnp.zeros((NPAD, D), F32)

    w1e = [p["proc"][s]["edge"]["layers"][0][0] for s in range(2)]
    b1e = [p["proc"][s]["edge"]["layers"][0][1] for s in range(2)]
    w1n = [p["proc"][s]["node"]["layers"][0][0] for s in range(2)]
    b1n = [p["proc"][s]["node"]["layers"][0][1] for s in range(2)]

    ctx_lat = _mlp_rows(context_features, p["enc_ctx"])
    node_lat, gsrc, gdst, bee, bne = _enc_node_pre(
        node_features, ctx_lat, p["enc_node"], w1e[0], b1e[0],
        w1n[0], b1n[0])

    gs, gd = _sc_gather(gsrc, gdst, src, dst)
    new_edge, edge_lat = _edge_step0(edge_features, gs, gd, p["enc_edge"],
                                     w1e[0], bee, p["proc"][0]["edge"])
    pooled2 = _sc_segment_sum(new_edge, dst, zeros)
    (node_lat, ctx_lat, gsrc, gdst, bee, bne) = _node_step_pre(
        node_lat, pooled2, ctx_lat, w1n[0], bne,
        p["proc"][0]["node"], p["proc"][0]["ctx"],
        w1e[1], b1e[1], w1n[1], b1n[1])

    gs, gd = _sc_gather(gsrc, gdst, src, dst)
    new_edge, edge_out = _edge_step1(edge_lat, gs, gd, w1e[1], bee,
                                     p["proc"][1]["edge"], p["dec_edge"])
    pooled2 = _sc_segment_sum(new_edge, dst, zeros)
    node_out, ctx_out = _node_step_dec(
        node_lat, pooled2, ctx_lat, w1n[1], bne,
        p["proc"][1]["node"], p["proc"][1]["ctx"],
        p["dec_node"], p["dec_ctx"])

    return (node_out, edge_out, ctx_out)
